# baseline (speedup 1.0000x reference)
import functools

import jax
import jax.numpy as jnp
from jax import lax
from jax.experimental import pallas as pl
from jax.experimental.pallas import tpu as pltpu
from jax.experimental.pallas import tpu_sc as plsc

N = 10000
E = 320000
M_PAD = 10240
NC, NS = 2, 16
NW = NC * NS
CH = 80
EPT = CH * 128
E_PAD = NW * EPT
RPT = M_PAD // NS
BM = 512
GRID = M_PAD // BM

_HI = jax.lax.Precision.HIGHEST


def _sc_mesh():
  return plsc.VectorSubcoreMesh(core_axis_name="c", subcore_axis_name="s")


@functools.partial(
    pl.kernel,
    out_type=jax.ShapeDtypeStruct((NC, NS, M_PAD), jnp.float32),
    mesh=_sc_mesh(),
    scratch_types=[
        pltpu.VMEM((EPT,), jnp.int32),
        pltpu.VMEM((M_PAD,), jnp.float32)
    ],
    compiler_params=pltpu.CompilerParams(needs_layout_passes=False),
)
def _deg_kernel(dstf_hbm, z_hbm, out_hbm, didx, hist):
  c = lax.axis_index("c")
  s = lax.axis_index("s")
  w = s * NC + c
  pltpu.sync_copy(z_hbm, hist)
  pltpu.sync_copy(dstf_hbm.at[w], didx)
  ones = jnp.ones((16,), jnp.float32)

  def body(i, carry):
    idx = didx[pl.ds(i * 16, 16)]
    plsc.addupdate_scatter(hist, [idx], ones)
    return carry

  lax.fori_loop(0, EPT // 16, body, 0)
  pltpu.sync_copy(hist, out_hbm.at[c, s])


NR = 4
CHF = 80
CHS = 80


def _make_agg(D):
  @functools.partial(
      pl.kernel,
      out_type=jax.ShapeDtypeStruct((NC, M_PAD, D), jnp.float32),
      mesh=_sc_mesh(),
      scratch_types=[
          pltpu.VMEM((CHF, 128), jnp.int32),
          [pltpu.VMEM((128,), jnp.int32)] * NR,
          [pltpu.VMEM((128, D), jnp.float32)] * 2,
          pltpu.VMEM_SHARED((M_PAD, D), jnp.float32),
          [pltpu.SemaphoreType.DMA] * NR,
          [pltpu.SemaphoreType.DMA] * 2,
          [pltpu.SemaphoreType.DMA] * 2,
      ],
  )
  def agg(y_hbm, src_hbm, dst_hbm, z_hbm, out_hbm, didx, iring, rows, acc,
          isem, gsem, ssem):
    c = lax.axis_index("c")
    s = lax.axis_index("s")
    r0 = s * RPT
    base = jnp.where(c == 1, s * CHF, NS * CHF + s * CHS)
    nch = jnp.where(c == 1, CHF, CHS)
    pltpu.sync_copy(z_hbm.at[pl.ds(r0, RPT)], acc.at[pl.ds(r0, RPT)])
    pltpu.sync_copy(dst_hbm.at[pl.ds(base, CHF)], didx)
    plsc.subcore_barrier()

    def i_load(j, q):
      pltpu.async_copy(src_hbm.at[base + j], iring[q], isem[q])

    def i_wait(q):
      pltpu.make_async_copy(src_hbm.at[0], iring[q], isem[q]).wait()

    def g_start(q, p):
      pltpu.async_copy(y_hbm.at[iring[q]], rows[p], gsem[p])

    def g_wait(p):
      pltpu.make_async_copy(y_hbm.at[iring[0]], rows[p], gsem[p]).wait()

    def s_start(j, p):
      pltpu.async_copy(rows[p], acc.at[didx.at[j]], ssem[p], add=True)

    def s_wait(p):
      pltpu.make_async_copy(rows[p], acc.at[pl.ds(0, 128)], ssem[p]).wait()

    def steady(j, b):
      q, p, p1, q1 = b, b % 2, 1 - (b % 2), (b - 1) % NR
      i_wait(q)
      s_wait(p)
      g_start(q, p)
      g_wait(p1)
      i_load(j - 1 + NR, q1)
      s_start(j - 1, p1)

    for q in range(NR):
      i_load(q, q)
    i_wait(0)
    g_start(0, 0)
    i_wait(1)
    g_start(1, 1)
    g_wait(0)
    i_load(NR, 0)
    s_start(0, 0)
    steady(2, 2)
    steady(3, 3)

    def round_body(r, carry):
      base = r * NR
      for b in range(NR):
        steady(base + b, b)
      return carry

    lax.fori_loop(1, nch // NR, round_body, 0)
    g_wait(1)
    s_start(nch - 1, 1)
    s_wait(0)
    s_wait(1)
    i_wait(0)
    i_wait(1)
    i_wait(2)
    plsc.subcore_barrier()
    pltpu.sync_copy(acc.at[pl.ds(r0, RPT)], out_hbm.at[c, pl.ds(r0, RPT)])

  return agg


_agg128 = _make_agg(128)


def _tc0_body(d_ref, x_ref, dis_ref, y_ref):
  d = d_ref[...]
  deg = jnp.sum(d, axis=(0, 1))[:, None] + 1.0
  dis = lax.rsqrt(deg)
  dis_ref[...] = jnp.broadcast_to(dis, dis_ref.shape)
  y_ref[...] = dis * x_ref[...]


def _tc0(d, x_pad):
  return pl.pallas_call(
      _tc0_body,
      grid=(GRID,),
      in_specs=[
          pl.BlockSpec((NC, NS, BM), lambda i: (0, 0, i)),
          pl.BlockSpec((BM, 128), lambda i: (i, 0)),
      ],
      out_specs=[
          pl.BlockSpec((BM, 128), lambda i: (i, 0)),
          pl.BlockSpec((BM, 128), lambda i: (i, 0)),
      ],
      out_shape=[
          jax.ShapeDtypeStruct((M_PAD, 128), jnp.float32),
          jax.ShapeDtypeStruct((M_PAD, 128), jnp.float32),
      ],
  )(d, x_pad)


def _tc1_body(s_ref, y_ref, dis_ref, w1_ref, b1_ref, w2_ref, out_ref):
  sblk = s_ref[...]
  t = dis_ref[...] * (sblk[0] + sblk[1] + y_ref[...])
  h = jnp.dot(t, w1_ref[...], precision=_HI,
              preferred_element_type=jnp.float32) + b1_ref[...]
  h = jnp.maximum(h, 0.0)
  out_ref[...] = dis_ref[...] * jnp.dot(
      h, w2_ref[...], precision=_HI, preferred_element_type=jnp.float32)


def _tc1(s1, y1, dis, W1, b1, W2):
  return pl.pallas_call(
      _tc1_body,
      grid=(GRID,),
      in_specs=[
          pl.BlockSpec((NC, BM, 128), lambda i: (0, i, 0)),
          pl.BlockSpec((BM, 128), lambda i: (i, 0)),
          pl.BlockSpec((BM, 128), lambda i: (i, 0)),
          pl.BlockSpec((128, 256), lambda i: (0, 0)),
          pl.BlockSpec((1, 256), lambda i: (0, 0)),
          pl.BlockSpec((256, 128), lambda i: (0, 0)),
      ],
      out_specs=pl.BlockSpec((BM, 128), lambda i: (i, 0)),
      out_shape=jax.ShapeDtypeStruct((M_PAD, 128), jnp.float32),
  )(s1, y1, dis, W1, b1, W2)


def _tc2_body(s_ref, y_ref, dis_ref, b2_ref, out_ref):
  sblk = s_ref[...]
  dis = dis_ref[...]
  h = dis * (sblk[0] + sblk[1] + y_ref[...]) + b2_ref[...]
  out_ref[...] = dis * jnp.maximum(h, 0.0)


def _tc2(s2, y2, dis, b2):
  return pl.pallas_call(
      _tc2_body,
      grid=(GRID,),
      in_specs=[
          pl.BlockSpec((NC, BM, 128), lambda i: (0, i, 0)),
          pl.BlockSpec((BM, 128), lambda i: (i, 0)),
          pl.BlockSpec((BM, 128), lambda i: (i, 0)),
          pl.BlockSpec((1, 128), lambda i: (0, 0)),
      ],
      out_specs=pl.BlockSpec((BM, 128), lambda i: (i, 0)),
      out_shape=jax.ShapeDtypeStruct((M_PAD, 128), jnp.float32),
  )(s2, y2, dis, b2)


def _tc3_body(s_ref, y_ref, dis_ref, w3_ref, b3_ref, out_ref):
  sblk = s_ref[...]
  a = dis_ref[...] * (sblk[0] + sblk[1] + y_ref[...])
  out_ref[...] = jnp.dot(
      a, w3_ref[...], precision=_HI,
      preferred_element_type=jnp.float32) + b3_ref[...]


def _tc3(s3, y3, dis, W3, b3):
  return pl.pallas_call(
      _tc3_body,
      grid=(GRID,),
      in_specs=[
          pl.BlockSpec((NC, BM, 128), lambda i: (0, i, 0)),
          pl.BlockSpec((BM, 128), lambda i: (i, 0)),
          pl.BlockSpec((BM, 128), lambda i: (i, 0)),
          pl.BlockSpec((128, 64), lambda i: (0, 0)),
          pl.BlockSpec((1, 64), lambda i: (0, 0)),
      ],
      out_specs=pl.BlockSpec((BM, 64), lambda i: (i, 0)),
      out_shape=jax.ShapeDtypeStruct((M_PAD, 64), jnp.float32),
  )(s3, y3, dis, W3, b3)


@jax.jit
def kernel(x, edge_index, W1, b1, W2, b2, W3, b3):
  pad = jnp.full((2, E_PAD - E), N, jnp.int32)
  ep = jnp.concatenate([edge_index, pad], axis=1)
  srcm = jnp.concatenate(
      [ep[0].reshape(NW * CH, 128), jnp.zeros((8, 128), jnp.int32)])
  dstm = jnp.concatenate(
      [ep[1].reshape(NW * CH, 128), jnp.full((CHF, 128), N, jnp.int32)])
  dstf = ep[1].reshape(NW, EPT)

  x_pad = jnp.zeros((M_PAD, 128), jnp.float32).at[:N].set(x)
  zrow = jnp.zeros((M_PAD,), jnp.float32)
  z128 = jnp.zeros((M_PAD, 128), jnp.float32)
  b1r = b1.reshape(1, -1)
  b2r = b2.reshape(1, -1)
  b3r = b3.reshape(1, -1)

  d = _deg_kernel(dstf, zrow)
  dis, y1 = _tc0(d, x_pad)
  s1 = _agg128(y1, srcm, dstm, z128)
  y2 = _tc1(s1, y1, dis, W1, b1r, W2)
  s2 = _agg128(y2, srcm, dstm, z128)
  y3 = _tc2(s2, y2, dis, b2r)
  s3 = _agg128(y3, srcm, dstm, z128)
  z_pad = _tc3(s3, y3, dis, W3, b3r)
  return z_pad[:N]


# speedup vs baseline: 1.2408x; 1.2408x over previous
import functools

import jax
import jax.numpy as jnp
from jax import lax
from jax.experimental import pallas as pl
from jax.experimental.pallas import tpu as pltpu
from jax.experimental.pallas import tpu_sc as plsc

N = 10000
E = 320000
M_PAD = 10240
NC, NS = 2, 16
NW = NC * NS
CH = 80
EPT = CH * 128
E_PAD = NW * EPT
RPT = M_PAD // NS
BM = 512
GRID = M_PAD // BM

_HI = jax.lax.Precision.HIGHEST


def _sc_mesh():
  return plsc.VectorSubcoreMesh(core_axis_name="c", subcore_axis_name="s")


@functools.partial(
    pl.kernel,
    out_type=jax.ShapeDtypeStruct((NC, NS, M_PAD), jnp.float32),
    mesh=_sc_mesh(),
    scratch_types=[
        pltpu.VMEM((EPT,), jnp.int32),
        pltpu.VMEM((M_PAD,), jnp.float32)
    ],
    compiler_params=pltpu.CompilerParams(needs_layout_passes=False),
)
def _deg_kernel(dstf_hbm, z_hbm, out_hbm, didx, hist):
  c = lax.axis_index("c")
  s = lax.axis_index("s")
  w = s * NC + c
  pltpu.sync_copy(z_hbm, hist)
  pltpu.sync_copy(dstf_hbm.at[w], didx)
  ones = jnp.ones((16,), jnp.float32)

  def body(i, carry):
    idx = didx[pl.ds(i * 16, 16)]
    plsc.addupdate_scatter(hist, [idx], ones)
    return carry

  lax.fori_loop(0, EPT // 16, body, 0)
  pltpu.sync_copy(hist, out_hbm.at[c, s])


NR = 4


def _make_agg(D):
  @functools.partial(
      pl.kernel,
      out_type=jax.ShapeDtypeStruct((NC, M_PAD, D), jnp.float32),
      mesh=_sc_mesh(),
      scratch_types=[
          pltpu.VMEM((CH, 128), jnp.int32),
          [pltpu.VMEM((128,), jnp.int32)] * NR,
          [pltpu.VMEM((128, D), jnp.float32)] * 2,
          pltpu.VMEM_SHARED((M_PAD, D), jnp.float32),
          [pltpu.SemaphoreType.DMA] * NR,
          [pltpu.SemaphoreType.DMA] * 2,
          [pltpu.SemaphoreType.DMA] * 2,
      ],
  )
  def agg(y_hbm, src_hbm, dst_hbm, z_hbm, out_hbm, didx, iring, rows, acc,
          isem, gsem, ssem):
    c = lax.axis_index("c")
    s = lax.axis_index("s")
    w = s * NC + c
    r0 = s * RPT
    pltpu.sync_copy(z_hbm.at[pl.ds(r0, RPT)], acc.at[pl.ds(r0, RPT)])
    pltpu.sync_copy(dst_hbm.at[pl.ds(w * CH, CH)], didx)
    plsc.subcore_barrier()

    def i_load(j, q):
      pltpu.async_copy(src_hbm.at[w * CH + j], iring[q], isem[q])

    def i_wait(q):
      pltpu.make_async_copy(src_hbm.at[0], iring[q], isem[q]).wait()

    def g_start(q, p):
      pltpu.async_copy(y_hbm.at[iring[q]], rows[p], gsem[p])

    def g_wait(p):
      pltpu.make_async_copy(y_hbm.at[iring[0]], rows[p], gsem[p]).wait()

    def s_start(j, p):
      pltpu.async_copy(rows[p], acc.at[didx.at[j]], ssem[p], add=True)

    def s_wait(p):
      pltpu.make_async_copy(rows[p], acc.at[pl.ds(0, 128)], ssem[p]).wait()

    def steady(j, b):
      q, p, p1, q1 = b, b % 2, 1 - (b % 2), (b - 1) % NR
      i_wait(q)
      s_wait(p)
      g_start(q, p)
      g_wait(p1)
      i_load(j - 1 + NR, q1)
      s_start(j - 1, p1)

    for q in range(NR):
      i_load(q, q)
    i_wait(0)
    g_start(0, 0)
    i_wait(1)
    g_start(1, 1)
    g_wait(0)
    i_load(NR, 0)
    s_start(0, 0)
    steady(2, 2)
    steady(3, 3)

    def round_body(r, carry):
      base = r * NR
      for b in range(NR):
        steady(base + b, b)
      return carry

    lax.fori_loop(1, CH // NR, round_body, 0)
    g_wait(1)
    s_start(CH - 1, 1)
    s_wait(0)
    s_wait(1)
    i_wait(0)
    i_wait(1)
    i_wait(2)
    plsc.subcore_barrier()
    pltpu.sync_copy(acc.at[pl.ds(r0, RPT)], out_hbm.at[c, pl.ds(r0, RPT)])

  return agg


_agg128 = _make_agg(128)


def _tc0_body(d_ref, x_ref, dis_ref, y_ref):
  d = d_ref[...]
  deg = jnp.sum(d, axis=(0, 1))[:, None] + 1.0
  dis = lax.rsqrt(deg)
  dis_ref[...] = jnp.broadcast_to(dis, dis_ref.shape)
  y_ref[...] = dis * x_ref[...]


def _tc0(d, x_pad):
  return pl.pallas_call(
      _tc0_body,
      grid=(GRID,),
      in_specs=[
          pl.BlockSpec((NC, NS, BM), lambda i: (0, 0, i)),
          pl.BlockSpec((BM, 128), lambda i: (i, 0)),
      ],
      out_specs=[
          pl.BlockSpec((BM, 128), lambda i: (i, 0)),
          pl.BlockSpec((BM, 128), lambda i: (i, 0)),
      ],
      out_shape=[
          jax.ShapeDtypeStruct((M_PAD, 128), jnp.float32),
          jax.ShapeDtypeStruct((M_PAD, 128), jnp.float32),
      ],
  )(d, x_pad)


def _tc1_body(s_ref, y_ref, dis_ref, w1_ref, b1_ref, w2_ref, out_ref):
  sblk = s_ref[...]
  t = dis_ref[...] * (sblk[0] + sblk[1] + y_ref[...])
  h = jnp.dot(t, w1_ref[...], precision=_HI,
              preferred_element_type=jnp.float32) + b1_ref[...]
  h = jnp.maximum(h, 0.0)
  out_ref[...] = dis_ref[...] * jnp.dot(
      h, w2_ref[...], precision=_HI, preferred_element_type=jnp.float32)


def _tc1(s1, y1, dis, W1, b1, W2):
  return pl.pallas_call(
      _tc1_body,
      grid=(GRID,),
      in_specs=[
          pl.BlockSpec((NC, BM, 128), lambda i: (0, i, 0)),
          pl.BlockSpec((BM, 128), lambda i: (i, 0)),
          pl.BlockSpec((BM, 128), lambda i: (i, 0)),
          pl.BlockSpec((128, 256), lambda i: (0, 0)),
          pl.BlockSpec((1, 256), lambda i: (0, 0)),
          pl.BlockSpec((256, 128), lambda i: (0, 0)),
      ],
      out_specs=pl.BlockSpec((BM, 128), lambda i: (i, 0)),
      out_shape=jax.ShapeDtypeStruct((M_PAD, 128), jnp.float32),
  )(s1, y1, dis, W1, b1, W2)


def _tc2_body(s_ref, y_ref, dis_ref, b2_ref, out_ref):
  sblk = s_ref[...]
  dis = dis_ref[...]
  h = dis * (sblk[0] + sblk[1] + y_ref[...]) + b2_ref[...]
  out_ref[...] = dis * jnp.maximum(h, 0.0)


def _tc2(s2, y2, dis, b2):
  return pl.pallas_call(
      _tc2_body,
      grid=(GRID,),
      in_specs=[
          pl.BlockSpec((NC, BM, 128), lambda i: (0, i, 0)),
          pl.BlockSpec((BM, 128), lambda i: (i, 0)),
          pl.BlockSpec((BM, 128), lambda i: (i, 0)),
          pl.BlockSpec((1, 128), lambda i: (0, 0)),
      ],
      out_specs=pl.BlockSpec((BM, 128), lambda i: (i, 0)),
      out_shape=jax.ShapeDtypeStruct((M_PAD, 128), jnp.float32),
  )(s2, y2, dis, b2)


def _tc3_body(s_ref, y_ref, dis_ref, w3_ref, b3_ref, out_ref):
  sblk = s_ref[...]
  a = dis_ref[...] * (sblk[0] + sblk[1] + y_ref[...])
  out_ref[...] = jnp.dot(
      a, w3_ref[...], precision=_HI,
      preferred_element_type=jnp.float32) + b3_ref[...]


def _tc3(s3, y3, dis, W3, b3):
  return pl.pallas_call(
      _tc3_body,
      grid=(GRID,),
      in_specs=[
          pl.BlockSpec((NC, BM, 128), lambda i: (0, i, 0)),
          pl.BlockSpec((BM, 128), lambda i: (i, 0)),
          pl.BlockSpec((BM, 128), lambda i: (i, 0)),
          pl.BlockSpec((128, 64), lambda i: (0, 0)),
          pl.BlockSpec((1, 64), lambda i: (0, 0)),
      ],
      out_specs=pl.BlockSpec((BM, 64), lambda i: (i, 0)),
      out_shape=jax.ShapeDtypeStruct((M_PAD, 64), jnp.float32),
  )(s3, y3, dis, W3, b3)


@jax.jit
def kernel(x, edge_index, W1, b1, W2, b2, W3, b3):
  pad = jnp.full((2, E_PAD - E), N, jnp.int32)
  ep = jnp.concatenate([edge_index, pad], axis=1)
  srcm = jnp.concatenate(
      [ep[0].reshape(NW * CH, 128), jnp.zeros((8, 128), jnp.int32)])
  dstm = ep[1].reshape(NW * CH, 128)
  dstf = ep[1].reshape(NW, EPT)

  x_pad = jnp.zeros((M_PAD, 128), jnp.float32).at[:N].set(x)
  zrow = jnp.zeros((M_PAD,), jnp.float32)
  z128 = jnp.zeros((M_PAD, 128), jnp.float32)
  b1r = b1.reshape(1, -1)
  b2r = b2.reshape(1, -1)
  b3r = b3.reshape(1, -1)

  d = _deg_kernel(dstf, zrow)
  dis, y1 = _tc0(d, x_pad)
  s1 = _agg128(y1, srcm, dstm, z128)
  y2 = _tc1(s1, y1, dis, W1, b1r, W2)
  s2 = _agg128(y2, srcm, dstm, z128)
  y3 = _tc2(s2, y2, dis, b2r)
  s3 = _agg128(y3, srcm, dstm, z128)
  z_pad = _tc3(s3, y3, dis, W3, b3r)
  return z_pad[:N]
